# baseline (speedup 1.0000x reference)
import functools

import numpy as np
import jax
import jax.numpy as jnp
from jax import lax
from jax.experimental import pallas as pl
from jax.experimental.pallas import tpu as pltpu
from jax.experimental.pallas import tpu_sc as plsc

B, X, C_IN = 8, 16, 32
K, S, F = 3, 2, 64
OX = (X - K) // S + 1
NPOS = OX * OX * OX
CK = K * K * K * C_IN
NTAP = K * K * K

NW = 32
ROWS = NPOS * B * NTAP
ROWS_PAD = 76032
RPW = ROWS_PAD // NW
NCH = 3
CH = RPW // NCH


def _sc_row_ids():
    idx = np.zeros((ROWS_PAD,), np.int32)
    r = 0
    for p in range(NPOS):
        x, y, z = p // (OX * OX), (p // OX) % OX, p % OX
        for b in range(B):
            for i in range(K):
                for j in range(K):
                    for l in range(K):
                        idx[r] = ((b * X + (S * x + i)) * X
                                  + (S * y + j)) * X + (S * z + l)
                        r += 1
    return idx.reshape(NW * NCH, 1, CH)


_IDX = _sc_row_ids()


def _sc_gather_body(tbl_hbm, stbl_hbm, idx_hbm, p_hbm, s_hbm,
                    idx_v, rows_v, srows_v, sem_p, sem_s):
    wid = lax.axis_index("s") * 2 + lax.axis_index("c")
    for c in range(NCH):
        base = wid * RPW + c * CH
        pltpu.sync_copy(idx_hbm.at[wid * NCH + c, 0], idx_v)
        cp_p = pltpu.async_copy(tbl_hbm.at[idx_v], rows_v, sem_p)
        cp_s = pltpu.async_copy(stbl_hbm.at[idx_v], srows_v, sem_s)
        cp_p.wait()
        cp_s.wait()
        pltpu.sync_copy(rows_v, p_hbm.at[pl.ds(base, CH)])
        pltpu.sync_copy(srows_v, s_hbm.at[pl.ds(base, CH)])


def _sc_gather(tbl, stbl, idx):
    mesh = plsc.VectorSubcoreMesh(core_axis_name="c", subcore_axis_name="s")
    k = functools.partial(
        pl.kernel,
        out_type=[jax.ShapeDtypeStruct((ROWS_PAD, C_IN), jnp.float32)] * 2,
        mesh=mesh,
        scratch_types=[
            pltpu.VMEM((CH,), jnp.int32),
            pltpu.VMEM((CH, C_IN), jnp.float32),
            pltpu.VMEM((CH, C_IN), jnp.float32),
            pltpu.SemaphoreType.DMA,
            pltpu.SemaphoreType.DMA,
        ],
        compiler_params=pltpu.CompilerParams(use_tc_tiling_on_sc=False),
    )(_sc_gather_body)
    return k(tbl, stbl, idx)


def _mm_body(p_ref, sp_ref, loc_ref, rho_ref, eps_ref, b_ref, so_ref, o_ref):
    for z in range(OX):
        p = p_ref[pl.ds(z * B, B)]
        ps = p * sp_ref[pl.ds(z * B, B)]
        loc = loc_ref[0, 0, z].reshape(CK, F)
        w2 = (jax.nn.softplus(rho_ref[0, 0, z].reshape(CK, F))
              * eps_ref[0, 0, z].reshape(CK, F))
        m = jnp.dot(p, loc, preferred_element_type=jnp.float32)
        pert = jnp.dot(ps, w2, preferred_element_type=jnp.float32)
        o_ref[z] = m + b_ref[0, 0, z][None, :] + pert * so_ref[:, 0, 0, z, :]


def kernel(inputs, kernel_loc, kernel_rho, bias, eps, sign_in, sign_out):
    tbl = inputs.reshape(B * X * X * X, C_IN)
    stbl = sign_in.reshape(B * X * X * X, C_IN)
    pout, sout = _sc_gather(tbl, stbl, jnp.asarray(_IDX))
    pv = pout.reshape(ROWS_PAD * C_IN // CK, CK)
    sv = sout.reshape(ROWS_PAD * C_IN // CK, CK)

    pspec = pl.BlockSpec((OX * B, CK), lambda i: (i, 0))
    wspec = pl.BlockSpec((1, 1, OX, K, K, K, C_IN, F),
                         lambda i: (i // OX, i % OX, 0, 0, 0, 0, 0, 0))
    out = pl.pallas_call(
        _mm_body,
        grid=(OX * OX,),
        in_specs=[
            pspec, pspec, wspec, wspec, wspec,
            pl.BlockSpec((1, 1, OX, F), lambda i: (i // OX, i % OX, 0, 0)),
            pl.BlockSpec((B, 1, 1, OX, F), lambda i: (0, i // OX, i % OX, 0, 0)),
        ],
        out_specs=pl.BlockSpec((OX, B, F), lambda i: (i, 0, 0)),
        out_shape=jax.ShapeDtypeStruct((NPOS, B, F), jnp.float32),
    )(pv, sv, kernel_loc, kernel_rho, eps, bias, sign_out)

    return out.reshape(OX, OX, OX, B, F).transpose(3, 0, 1, 2, 4)
